# baseline (speedup 1.0000x reference)
import functools

import jax
import jax.numpy as jnp
from jax import lax
from jax.experimental import pallas as pl
from jax.experimental.pallas import tpu as pltpu
from jax.experimental.pallas import tpu_sc as plsc

NUM_ROWS = 1_000_000
DIM = 64
SCALE = 8.0

NC = 2
NS = 16
NW = NC * NS


_TC_BLOCK_ROWS = 2000


def _normalize_body(t_ref, o_ref):
    v = t_ref[...]
    s = jnp.sum(v * v, axis=-1, keepdims=True)
    o_ref[...] = v * (SCALE / jnp.maximum(jnp.sqrt(s), 1e-12))


def _normalize_table(table):
    grid = NUM_ROWS // _TC_BLOCK_ROWS
    return pl.pallas_call(
        _normalize_body,
        grid=(grid,),
        in_specs=[pl.BlockSpec((_TC_BLOCK_ROWS, DIM), lambda i: (i, 0))],
        out_specs=pl.BlockSpec((_TC_BLOCK_ROWS, DIM), lambda i: (i, 0)),
        out_shape=jax.ShapeDtypeStruct((NUM_ROWS, DIM), jnp.float32),
    )(table)


_CHUNK = 512
_SUB = 128


def _make_sc_gather(batch):
    assert batch % (NW * _CHUNK) == 0
    b_per_w = batch // NW
    n_chunks = b_per_w // _CHUNK
    mesh = plsc.VectorSubcoreMesh(core_axis_name="c", subcore_axis_name="s")

    @functools.partial(
        pl.kernel,
        mesh=mesh,
        compiler_params=pltpu.CompilerParams(use_tc_tiling_on_sc=False),
        out_type=jax.ShapeDtypeStruct((batch, DIM), jnp.float32),
        scratch_types=[
            pltpu.VMEM((_CHUNK,), jnp.int32),
            pltpu.VMEM((_CHUNK, DIM), jnp.float32),
            pltpu.SemaphoreType.DMA,
        ],
    )
    def gather_kernel(table_hbm, idx_hbm, out_hbm, idx_v, rows_v, sem):
        wid = lax.axis_index("s") * NC + lax.axis_index("c")
        base0 = wid * b_per_w

        def body(g, carry):
            base = base0 + g * _CHUNK
            pltpu.sync_copy(idx_hbm.at[pl.ds(base, _CHUNK)], idx_v)
            copies = [
                pltpu.async_copy(
                    table_hbm.at[idx_v.at[pl.ds(j * _SUB, _SUB)]],
                    rows_v.at[pl.ds(j * _SUB, _SUB)],
                    sem,
                )
                for j in range(_CHUNK // _SUB)
            ]
            for cp in copies:
                cp.wait()
            pltpu.sync_copy(rows_v, out_hbm.at[pl.ds(base, _CHUNK)])
            return carry

        lax.fori_loop(0, n_chunks, body, 0)

    return gather_kernel


def kernel(x, raw_embedding):
    idx = x.reshape(-1).astype(jnp.int32)
    batch = idx.shape[0]
    norm_table = _normalize_table(raw_embedding)
    rows = _make_sc_gather(batch)(norm_table, idx)
    return rows.reshape(x.shape + (DIM,))


# speedup vs baseline: 1.1409x; 1.1409x over previous
import functools

import jax
import jax.numpy as jnp
from jax import lax
from jax.experimental import pallas as pl
from jax.experimental.pallas import tpu as pltpu
from jax.experimental.pallas import tpu_sc as plsc

NUM_ROWS = 1_000_000
DIM = 64
SCALE = 8.0

NC = 2
NS = 16
NW = NC * NS


_TC_BLOCK_ROWS = 2000


def _normalize_body(t_ref, o_ref):
    v = t_ref[...]
    s = jnp.sum(v * v, axis=-1, keepdims=True)
    o_ref[...] = v * (SCALE / jnp.maximum(jnp.sqrt(s), 1e-12))


def _normalize_table(table):
    grid = NUM_ROWS // _TC_BLOCK_ROWS
    return pl.pallas_call(
        _normalize_body,
        grid=(grid,),
        in_specs=[pl.BlockSpec((_TC_BLOCK_ROWS, DIM), lambda i: (i, 0))],
        out_specs=pl.BlockSpec((_TC_BLOCK_ROWS, DIM), lambda i: (i, 0)),
        out_shape=jax.ShapeDtypeStruct((NUM_ROWS, DIM), jnp.float32),
    )(table)


_CHUNK = 512
_SUB = 128


def _make_sc_gather(batch):
    assert batch % (NW * _CHUNK) == 0
    b_per_w = batch // NW
    n_chunks = b_per_w // _CHUNK
    mesh = plsc.VectorSubcoreMesh(core_axis_name="c", subcore_axis_name="s")

    @functools.partial(
        pl.kernel,
        mesh=mesh,
        compiler_params=pltpu.CompilerParams(
            use_tc_tiling_on_sc=False, needs_layout_passes=False
        ),
        out_type=jax.ShapeDtypeStruct((batch, DIM), jnp.float32),
        scratch_types=[
            pltpu.VMEM((_CHUNK,), jnp.int32),
            pltpu.VMEM((_CHUNK, DIM), jnp.float32),
            pltpu.VMEM((272,), jnp.float32),
            pltpu.VMEM((272,), jnp.float32),
            pltpu.SemaphoreType.DMA,
        ],
    )
    def gather_kernel(table_hbm, idx_hbm, out_hbm, idx_v, rows_v, tr_v, fs_v, sem):
        wid = lax.axis_index("s") * NC + lax.axis_index("c")
        base0 = wid * b_per_w

        def body(g, carry):
            base = base0 + g * _CHUNK
            pltpu.sync_copy(idx_hbm.at[pl.ds(base, _CHUNK)], idx_v)
            copies = [
                pltpu.async_copy(
                    table_hbm.at[idx_v.at[pl.ds(j * _SUB, _SUB)]],
                    rows_v.at[pl.ds(j * _SUB, _SUB)],
                    sem,
                )
                for j in range(_CHUNK // _SUB)
            ]
            for cp in copies:
                cp.wait()
            _normalize_chunk(rows_v, tr_v, fs_v)
            pltpu.sync_copy(rows_v, out_hbm.at[pl.ds(base, _CHUNK)])
            return carry

        lax.fori_loop(0, n_chunks, body, 0)

    return gather_kernel


_MAGIC = 0x5F3759DF


def _normalize_chunk(rows_v, tr_v, fs_v):
    lanes17 = lax.iota(jnp.int32, 16) * 17

    def group(i, carry):
        r0 = i * 16
        for j in range(16):
            r = r0 + j
            v = [rows_v[r, pl.ds(m * 16, 16)] for m in range(4)]
            t = v[0] * v[0]
            for m in range(1, 4):
                t = v[m] * v[m] + t
            plsc.store_scatter(tr_v, [lanes17 + j], t)
        s = tr_v[pl.ds(0, 16)]
        for l in range(1, 16):
            s = s + tr_v[pl.ds(l * 17, 16)]
        sv = jnp.maximum(s, 1e-24)
        i32 = lax.bitcast_convert_type(sv, jnp.int32)
        y = lax.bitcast_convert_type(
            jnp.int32(_MAGIC) - lax.shift_right_logical(i32, 1), jnp.float32
        )
        for _ in range(3):
            y = y * (1.5 - 0.5 * sv * y * y)
        f = y * SCALE
        for l in range(16):
            plsc.store_scatter(fs_v, [lanes17 + l], f)
        for j in range(16):
            r = r0 + j
            fj = fs_v[pl.ds(j * 17, 16)]
            for m in range(4):
                rows_v[r, pl.ds(m * 16, 16)] = rows_v[r, pl.ds(m * 16, 16)] * fj
        return carry

    lax.fori_loop(0, _CHUNK // 16, group, 0)


def kernel(x, raw_embedding):
    idx = x.reshape(-1).astype(jnp.int32)
    batch = idx.shape[0]
    rows = _make_sc_gather(batch)(raw_embedding, idx)
    return rows.reshape(x.shape + (DIM,))


# speedup vs baseline: 1.4347x; 1.2575x over previous
import functools

import jax
import jax.numpy as jnp
from jax import lax
from jax.experimental import pallas as pl
from jax.experimental.pallas import tpu as pltpu
from jax.experimental.pallas import tpu_sc as plsc

DIM = 64
SCALE = 8.0
_MAGIC = 0x5F3759DF

NC = 2
NS = 16
NW = NC * NS

_XROWS_PER_CHUNK = 2
_SPLITS = (0, 104)


def _make_sc_kernel(n_x, n_inner):
    assert n_x % NW == 0 and n_x // NW % _XROWS_PER_CHUNK == 0
    rows_per_w = n_x // NW
    n_chunks = rows_per_w // _XROWS_PER_CHUNK
    assert n_chunks % 4 == 0
    chunk_rows = _XROWS_PER_CHUNK * n_inner
    assert chunk_rows % 16 == 0
    mesh = plsc.VectorSubcoreMesh(core_axis_name="c", subcore_axis_name="s")

    @functools.partial(
        pl.kernel,
        mesh=mesh,
        compiler_params=pltpu.CompilerParams(
            use_tc_tiling_on_sc=False, needs_layout_passes=False
        ),
        out_type=jax.ShapeDtypeStruct((n_x, n_inner, DIM), jnp.float32),
        scratch_types=[
            pltpu.VMEM((_XROWS_PER_CHUNK, n_inner), jnp.int32),
            pltpu.VMEM((_XROWS_PER_CHUNK, n_inner), jnp.int32),
            pltpu.VMEM((_XROWS_PER_CHUNK, n_inner), jnp.int32),
            pltpu.VMEM((_XROWS_PER_CHUNK, n_inner), jnp.int32),
            pltpu.VMEM((chunk_rows, DIM), jnp.float32),
            pltpu.VMEM((chunk_rows, DIM), jnp.float32),
            pltpu.VMEM((chunk_rows, DIM), jnp.float32),
            pltpu.VMEM((chunk_rows, DIM), jnp.float32),
            pltpu.SemaphoreType.DMA,
        ],
    )
    def sc_kernel(
        table_hbm, idx_hbm, out_hbm,
        idx_v0, idx_v1, idx_v2, idx_v3,
        rows_v0, rows_v1, rows_v2, rows_v3, sem,
    ):
        wid = lax.axis_index("s") * NC + lax.axis_index("c")
        row_base = wid * rows_per_w
        idx_v = (idx_v0, idx_v1, idx_v2, idx_v3)
        rows_v = (rows_v0, rows_v1, rows_v2, rows_v3)

        def start_gather(b, x0):
            pltpu.sync_copy(
                idx_hbm.at[pl.ds(x0, _XROWS_PER_CHUNK)], idx_v[b]
            )
            cps = []
            for a in range(_XROWS_PER_CHUNK):
                for h, lo in enumerate(_SPLITS):
                    n = (_SPLITS + (n_inner,))[h + 1] - lo
                    cps.append(pltpu.async_copy(
                        table_hbm.at[idx_v[b].at[a, pl.ds(lo, n)]],
                        rows_v[b].at[pl.ds(a * n_inner + lo, n)],
                        sem,
                    ))
            return cps

        def writeback(b, x0):
            for a in range(_XROWS_PER_CHUNK):
                pltpu.sync_copy(
                    rows_v[b].at[pl.ds(a * n_inner, n_inner)],
                    out_hbm.at[x0 + a],
                )

        def body(k, carry):
            xs = [row_base + (4 * k + b) * _XROWS_PER_CHUNK for b in range(4)]
            cps = start_gather(0, xs[0])
            for b in range(4):
                for cp in cps:
                    cp.wait()
                if b < 3:
                    cps = start_gather(b + 1, xs[b + 1])
                _normalize_chunk(rows_v[b], chunk_rows)
                writeback(b, xs[b])
            return carry

        lax.fori_loop(0, n_chunks // 4, body, 0)

    return sc_kernel


def _normalize_chunk(rows_v, chunk_rows):
    def group(i, carry):
        r0 = i * 16
        for j in range(16):
            r = r0 + j
            v = [rows_v[r, pl.ds(m * 16, 16)] for m in range(4)]
            t = v[0] * v[0]
            for m in range(1, 4):
                t = v[m] * v[m] + t
            s = jnp.maximum(jnp.sum(t), 1e-24)
            si = lax.bitcast_convert_type(s, jnp.int32)
            y = lax.bitcast_convert_type(
                jnp.int32(_MAGIC) - lax.shift_right_logical(si, 1), jnp.float32
            )
            for _ in range(2):
                y = y * (1.5 - 0.5 * s * y * y)
            f = jnp.full((16,), y * SCALE, jnp.float32)
            for m in range(4):
                rows_v[r, pl.ds(m * 16, 16)] = v[m] * f
        return carry

    lax.fori_loop(0, chunk_rows // 16, group, 0)


def kernel(x, raw_embedding):
    idx = x.astype(jnp.int32)
    n_x, n_inner = idx.shape
    return _make_sc_kernel(n_x, n_inner)(raw_embedding, idx)
